# Initial kernel scaffold; baseline (speedup 1.0000x reference)
import jax
import jax.numpy as jnp
from jax.experimental import pallas as pl


def kernel(x, Embeddings):
    raise NotImplementedError("write your pallas kernel here")


# speedup vs baseline: 1.6846x; 1.6846x over previous
import functools

import jax
import jax.numpy as jnp
from jax import lax
from jax.experimental import pallas as pl
from jax.experimental.pallas import tpu as pltpu
from jax.experimental.pallas import tpu_sc as plsc

NC = 2
NS = 16
NW = NC * NS
D = 64
C = 128


@functools.partial(jax.jit, static_argnames=("b_per_w",))
def _emb_lookup(idx3, table, *, b_per_w):
    n_chunks = b_per_w // C
    B = NW * b_per_w

    mesh = plsc.VectorSubcoreMesh(core_axis_name="c", subcore_axis_name="s")

    @functools.partial(
        pl.kernel,
        out_type=jax.ShapeDtypeStruct((B, D), jnp.float32),
        mesh=mesh,
        scratch_types=[
            pltpu.VMEM((n_chunks, C), jnp.int32),
            pltpu.VMEM((C, D), jnp.float32),
            pltpu.SemaphoreType.DMA,
        ],
        compiler_params=pltpu.CompilerParams(use_tc_tiling_on_sc=False),
    )
    def emb(table_hbm, idx_hbm, out_hbm, idx_v, rows_v, sem):
        wid = lax.axis_index("s") * NC + lax.axis_index("c")
        base = wid * b_per_w
        pltpu.sync_copy(idx_hbm.at[wid], idx_v)

        def body(j, carry):
            pltpu.async_copy(table_hbm.at[idx_v.at[j]], rows_v, sem).wait()
            pltpu.sync_copy(rows_v, out_hbm.at[pl.ds(base + j * C, C)])
            return carry

        lax.fori_loop(0, n_chunks, body, 0)

    return emb(table, idx3)


def kernel(x, Embeddings):
    B0, B1 = x.shape
    B = B0 * B1
    b_per_w = B // NW
    idx3 = x.astype(jnp.int32).reshape(NW, b_per_w // C, C)
    out = _emb_lookup(idx3, Embeddings, b_per_w=b_per_w)
    return out.reshape(B0, B1, D)


# speedup vs baseline: 1.8896x; 1.1217x over previous
import functools

import jax
import jax.numpy as jnp
from jax import lax
from jax.experimental import pallas as pl
from jax.experimental.pallas import tpu as pltpu
from jax.experimental.pallas import tpu_sc as plsc

NC = 2
NS = 16
NW = NC * NS
D = 64
C = 128


@functools.partial(jax.jit, static_argnames=("b_per_w",))
def _emb_lookup(idx3, table, *, b_per_w):
    n_chunks = b_per_w // C
    B = NW * b_per_w

    mesh = plsc.VectorSubcoreMesh(core_axis_name="c", subcore_axis_name="s")

    NB = 4

    @functools.partial(
        pl.kernel,
        out_type=jax.ShapeDtypeStruct((B, D), jnp.float32),
        mesh=mesh,
        scratch_types=[
            pltpu.VMEM((n_chunks, C), jnp.int32),
            pltpu.VMEM((NB, C, D), jnp.float32),
            pltpu.SemaphoreType.DMA,
            pltpu.SemaphoreType.DMA,
        ],
        compiler_params=pltpu.CompilerParams(use_tc_tiling_on_sc=False),
    )
    def emb(table_hbm, idx_hbm, out_hbm, idx_v, rows_v, sem_g, sem_s):
        wid = lax.axis_index("s") * NC + lax.axis_index("c")
        base = wid * b_per_w
        pltpu.sync_copy(idx_hbm.at[wid], idx_v)

        def gather(j, b):
            return pltpu.async_copy(table_hbm.at[idx_v.at[j]], rows_v.at[b], sem_g)

        for b in range(NB):
            gather(b, b)

        def group(g0s, carry):
            g0 = g0s * NB
            for b in range(NB):
                g = g0 + b
                pltpu.make_async_copy(
                    table_hbm.at[idx_v.at[g]], rows_v.at[b], sem_g
                ).wait()
                pltpu.async_copy(
                    rows_v.at[b], out_hbm.at[pl.ds(base + g * C, C)], sem_s
                )
                pltpu.make_async_copy(
                    rows_v.at[b], out_hbm.at[pl.ds(base + g * C, C)], sem_s
                ).wait()

                @pl.when(g + NB < n_chunks)
                def _():
                    gather(g + NB, b)

            return carry

        lax.fori_loop(0, n_chunks // NB, group, 0)

    return emb(table, idx3)


def kernel(x, Embeddings):
    B0, B1 = x.shape
    B = B0 * B1
    b_per_w = B // NW
    idx3 = x.astype(jnp.int32).reshape(NW, b_per_w // C, C)
    out = _emb_lookup(idx3, Embeddings, b_per_w=b_per_w)
    return out.reshape(B0, B1, D)
